# baseline (speedup 1.0000x reference)
import functools

import jax
import jax.numpy as jnp
from jax import lax
from jax.experimental import pallas as pl
from jax.experimental.pallas import tpu as pltpu
from jax.experimental.pallas import tpu_sc as plsc

_NUM_WORKERS = 32
_CHUNK = 128


def _sc_gather_pair(urow, mrow, Eu2, Em2):
  nrows_total, W = Eu2.shape[0], Eu2.shape[1]
  B = urow.shape[0] * urow.shape[1]
  rows_w = B // _NUM_WORKERS
  nck = rows_w // _CHUNK

  mesh = plsc.VectorSubcoreMesh(core_axis_name="c", subcore_axis_name="s")

  @functools.partial(
      pl.kernel,
      out_type=(jax.ShapeDtypeStruct((B, W), jnp.float32),
                jax.ShapeDtypeStruct((B, W), jnp.float32)),
      mesh=mesh,
      compiler_params=pltpu.CompilerParams(use_tc_tiling_on_sc=False),
      scratch_types=[
          pltpu.VMEM((nck, _CHUNK), jnp.int32),
          pltpu.VMEM((nck, _CHUNK), jnp.int32),
          pltpu.VMEM((rows_w, W), jnp.float32),
          pltpu.SemaphoreType.DMA,
          pltpu.SemaphoreType.DMA,
      ],
  )
  def gather_kernel(u_hbm, m_hbm, eu_hbm, em_hbm, ou_hbm, om_hbm,
                    iu_v, im_v, rows_v, sem_u, sem_m):
    wid = lax.axis_index("s") * 2 + lax.axis_index("c")
    base_ck = wid * nck
    base = wid * rows_w
    pltpu.sync_copy(u_hbm.at[pl.ds(base_ck, nck)], iu_v)
    pltpu.sync_copy(m_hbm.at[pl.ds(base_ck, nck)], im_v)
    copies = []
    for j in range(nck):
      copies.append(pltpu.async_copy(
          eu_hbm.at[iu_v.at[j]], rows_v.at[pl.ds(j * _CHUNK, _CHUNK)], sem_u))
    for c in copies:
      c.wait()
    pltpu.sync_copy(rows_v, ou_hbm.at[pl.ds(base, rows_w)])
    copies = []
    for j in range(nck):
      copies.append(pltpu.async_copy(
          em_hbm.at[im_v.at[j]], rows_v.at[pl.ds(j * _CHUNK, _CHUNK)], sem_m))
    for c in copies:
      c.wait()
    pltpu.sync_copy(rows_v, om_hbm.at[pl.ds(base, rows_w)])

  return gather_kernel(urow, mrow, Eu2, Em2)


def _tc_dense(gender, age, job, upar, mpar, blku, blkm,
              E_g, E_a, E_j, att_W, att_b):
  B = blku.shape[0]
  D = att_W.shape[1]
  BM = 1024
  NB = B // BM

  def pad_rows(t, n):
    return jnp.concatenate(
        [t, jnp.zeros((n - t.shape[0], t.shape[1]), t.dtype)], axis=0)

  NG, NA, NJ = 8, 8, 24
  Egp = pad_rows(E_g, NG)
  Eap = pad_rows(E_a, NA)
  Ejp = pad_rows(E_j, NJ)
  g3 = gender.reshape(NB, BM, 1)
  a3 = age.reshape(NB, BM, 1)
  j3 = job.reshape(NB, BM, 1)
  up3 = upar.reshape(NB, BM, 1)
  mp3 = mpar.reshape(NB, BM, 1)
  b2 = att_b.reshape(1, D)

  def body(g_ref, a_ref, j_ref, up_ref, mp_ref, bu_ref, bm_ref,
           eg_ref, ea_ref, ej_ref, w_ref, b_ref, o_ref):
    usel = up_ref[0] == 1
    msel = mp_ref[0] == 1
    bu = bu_ref[...]
    bm = bm_ref[...]
    eu = jnp.where(usel, bu[:, D:], bu[:, :D])
    em = jnp.where(msel, bm[:, D:], bm[:, :D])
    w_top = w_ref[:D, :]
    w_bot = w_ref[D:, :]
    m = jnp.dot(em, w_top, preferred_element_type=jnp.float32) + b_ref[...]

    def attr_cell(idx_ref, table_ref, n):
      ids = idx_ref[0]
      oh = (ids == lax.broadcasted_iota(jnp.int32, (BM, n), 1)
            ).astype(jnp.float32)
      tbl = table_ref[...]
      proj = jnp.dot(tbl, w_bot, preferred_element_type=jnp.float32)
      both = jnp.dot(oh, jnp.concatenate([tbl, proj], axis=1),
                     preferred_element_type=jnp.float32)
      e_att = both[:, :D]
      v = m + both[:, D:]
      v = v - jnp.max(v, axis=1, keepdims=True)
      ex = jnp.exp(v)
      lam = ex / jnp.sum(ex, axis=1, keepdims=True)
      return lam * e_att

    cg = attr_cell(g_ref, eg_ref, NG)
    ca = attr_cell(a_ref, ea_ref, NA)
    cj = attr_cell(j_ref, ej_ref, NJ)
    t = cg + ca + cj
    mn = cg * cg + ca * ca + cj * cj
    p_u = eu * t + 0.5 * (t * t - mn)
    o_ref[...] = jnp.sum(p_u * em, axis=1, keepdims=True)

  idx_spec = pl.BlockSpec((1, BM, 1), lambda i: (i, 0, 0))
  return pl.pallas_call(
      body,
      grid=(NB,),
      in_specs=[
          idx_spec, idx_spec, idx_spec, idx_spec, idx_spec,
          pl.BlockSpec((BM, 2 * D), lambda i: (i, 0)),
          pl.BlockSpec((BM, 2 * D), lambda i: (i, 0)),
          pl.BlockSpec((NG, D), lambda i: (0, 0)),
          pl.BlockSpec((NA, D), lambda i: (0, 0)),
          pl.BlockSpec((NJ, D), lambda i: (0, 0)),
          pl.BlockSpec((2 * D, D), lambda i: (0, 0)),
          pl.BlockSpec((1, D), lambda i: (0, 0)),
      ],
      out_specs=pl.BlockSpec((BM, 1), lambda i: (i, 0)),
      out_shape=jax.ShapeDtypeStruct((B, 1), jnp.float32),
  )(g3, a3, j3, up3, mp3, blku, blkm, Egp, Eap, Ejp, att_W, b2)


def kernel(uid, gender, age, job, mid, E_uid, E_gender, E_age, E_job, E_mid,
           att_W, att_b):
  B = uid.shape[0]
  Eu2 = E_uid.reshape(E_uid.shape[0] // 2, 2 * E_uid.shape[1])
  Em2 = E_mid.reshape(E_mid.shape[0] // 2, 2 * E_mid.shape[1])
  urow = jnp.right_shift(uid, 1).reshape(B // _CHUNK, _CHUNK)
  mrow = jnp.right_shift(mid, 1).reshape(B // _CHUNK, _CHUNK)
  upar = jnp.bitwise_and(uid, 1)
  mpar = jnp.bitwise_and(mid, 1)
  blku, blkm = _sc_gather_pair(urow, mrow, Eu2, Em2)
  return _tc_dense(gender, age, job, upar, mpar, blku, blkm,
                   E_gender, E_age, E_job, att_W, att_b)


# speedup vs baseline: 1.5604x; 1.5604x over previous
import functools

import jax
import jax.numpy as jnp
from jax import lax
from jax.experimental import pallas as pl
from jax.experimental.pallas import tpu as pltpu
from jax.experimental.pallas import tpu_sc as plsc

_NUM_WORKERS = 32
_CHUNK = 128
_TBLK = 2048


def _tc_transpose_pack(eT):
  D, N = eT.shape
  grid = pl.cdiv(N, 2 * _TBLK)
  last_blk = (N - 1) // _TBLK

  def body(xa_ref, xb_ref, o_ref):
    ya = jnp.transpose(xa_ref[...], (1, 0))
    yb = jnp.transpose(xb_ref[...], (1, 0))
    o_ref[...] = jnp.concatenate([ya, yb], axis=1)

  return pl.pallas_call(
      body,
      grid=(grid,),
      in_specs=[
          pl.BlockSpec((D, _TBLK),
                       lambda i: (0, jnp.minimum(2 * i, last_blk))),
          pl.BlockSpec((D, _TBLK),
                       lambda i: (0, jnp.minimum(2 * i + 1, last_blk))),
      ],
      out_specs=pl.BlockSpec((_TBLK, 2 * D), lambda i: (i, 0)),
      out_shape=jax.ShapeDtypeStruct((grid * _TBLK, 2 * D), jnp.float32),
  )(eT, eT)


def _sc_gather_pair(urow, mrow, Eu2, Em2):
  W = Eu2.shape[1]
  B = urow.shape[0] * urow.shape[1]
  rows_w = B // _NUM_WORKERS
  nck = rows_w // _CHUNK

  mesh = plsc.VectorSubcoreMesh(core_axis_name="c", subcore_axis_name="s")

  @functools.partial(
      pl.kernel,
      out_type=(jax.ShapeDtypeStruct((B, W), jnp.float32),
                jax.ShapeDtypeStruct((B, W), jnp.float32)),
      mesh=mesh,
      compiler_params=pltpu.CompilerParams(use_tc_tiling_on_sc=False),
      scratch_types=[
          pltpu.VMEM((nck, _CHUNK), jnp.int32),
          pltpu.VMEM((nck, _CHUNK), jnp.int32),
          pltpu.VMEM((rows_w, W), jnp.float32),
          pltpu.SemaphoreType.DMA,
          pltpu.SemaphoreType.DMA,
      ],
  )
  def gather_kernel(u_hbm, m_hbm, eu_hbm, em_hbm, ou_hbm, om_hbm,
                    iu_v, im_v, rows_v, sem_u, sem_m):
    wid = lax.axis_index("s") * 2 + lax.axis_index("c")
    base_ck = wid * nck
    base = wid * rows_w
    pltpu.sync_copy(u_hbm.at[pl.ds(base_ck, nck)], iu_v)
    pltpu.sync_copy(m_hbm.at[pl.ds(base_ck, nck)], im_v)
    copies = []
    for j in range(nck):
      copies.append(pltpu.async_copy(
          eu_hbm.at[iu_v.at[j]], rows_v.at[pl.ds(j * _CHUNK, _CHUNK)], sem_u))
    for c in copies:
      c.wait()
    pltpu.sync_copy(rows_v, ou_hbm.at[pl.ds(base, rows_w)])
    copies = []
    for j in range(nck):
      copies.append(pltpu.async_copy(
          em_hbm.at[im_v.at[j]], rows_v.at[pl.ds(j * _CHUNK, _CHUNK)], sem_m))
    for c in copies:
      c.wait()
    pltpu.sync_copy(rows_v, om_hbm.at[pl.ds(base, rows_w)])

  return gather_kernel(urow, mrow, Eu2, Em2)


def _tc_dense(gender, age, job, upar, mpar, blku, blkm,
              E_g, E_a, E_j, att_W, att_b):
  B = blku.shape[0]
  D = att_W.shape[1]
  BM = 1024
  NB = B // BM

  def pad_rows(t, n):
    return jnp.concatenate(
        [t, jnp.zeros((n - t.shape[0], t.shape[1]), t.dtype)], axis=0)

  NG, NA, NJ = 8, 8, 24
  Egp = pad_rows(E_g, NG)
  Eap = pad_rows(E_a, NA)
  Ejp = pad_rows(E_j, NJ)
  g3 = gender.reshape(NB, BM, 1)
  a3 = age.reshape(NB, BM, 1)
  j3 = job.reshape(NB, BM, 1)
  up3 = upar.reshape(NB, BM, 1)
  mp3 = mpar.reshape(NB, BM, 1)
  b2 = att_b.reshape(1, D)

  def body(g_ref, a_ref, j_ref, up_ref, mp_ref, bu_ref, bm_ref,
           eg_ref, ea_ref, ej_ref, w_ref, b_ref, o_ref):
    usel = up_ref[0] == 1
    msel = mp_ref[0] == 1
    bu = bu_ref[...]
    bm = bm_ref[...]
    eu = jnp.where(usel, bu[:, D:], bu[:, :D])
    em = jnp.where(msel, bm[:, D:], bm[:, :D])
    w_top = w_ref[:D, :]
    w_bot = w_ref[D:, :]
    m = jnp.dot(em, w_top, preferred_element_type=jnp.float32) + b_ref[...]

    def attr_cell(idx_ref, table_ref, n):
      ids = idx_ref[0]
      oh = (ids == lax.broadcasted_iota(jnp.int32, (BM, n), 1)
            ).astype(jnp.float32)
      tbl = table_ref[...]
      proj = jnp.dot(tbl, w_bot, preferred_element_type=jnp.float32)
      both = jnp.dot(oh, jnp.concatenate([tbl, proj], axis=1),
                     preferred_element_type=jnp.float32)
      e_att = both[:, :D]
      v = m + both[:, D:]
      v = v - jnp.max(v, axis=1, keepdims=True)
      ex = jnp.exp(v)
      lam = ex / jnp.sum(ex, axis=1, keepdims=True)
      return lam * e_att

    cg = attr_cell(g_ref, eg_ref, NG)
    ca = attr_cell(a_ref, ea_ref, NA)
    cj = attr_cell(j_ref, ej_ref, NJ)
    t = cg + ca + cj
    mn = cg * cg + ca * ca + cj * cj
    p_u = eu * t + 0.5 * (t * t - mn)
    o_ref[...] = jnp.sum(p_u * em, axis=1, keepdims=True)

  idx_spec = pl.BlockSpec((1, BM, 1), lambda i: (i, 0, 0))
  return pl.pallas_call(
      body,
      grid=(NB,),
      in_specs=[
          idx_spec, idx_spec, idx_spec, idx_spec, idx_spec,
          pl.BlockSpec((BM, 2 * D), lambda i: (i, 0)),
          pl.BlockSpec((BM, 2 * D), lambda i: (i, 0)),
          pl.BlockSpec((NG, D), lambda i: (0, 0)),
          pl.BlockSpec((NA, D), lambda i: (0, 0)),
          pl.BlockSpec((NJ, D), lambda i: (0, 0)),
          pl.BlockSpec((2 * D, D), lambda i: (0, 0)),
          pl.BlockSpec((1, D), lambda i: (0, 0)),
      ],
      out_specs=pl.BlockSpec((BM, 1), lambda i: (i, 0)),
      out_shape=jax.ShapeDtypeStruct((B, 1), jnp.float32),
  )(g3, a3, j3, up3, mp3, blku, blkm, Egp, Eap, Ejp, att_W, b2)


def kernel(uid, gender, age, job, mid, E_uid, E_gender, E_age, E_job, E_mid,
           att_W, att_b):
  B = uid.shape[0]
  Eu2 = _tc_transpose_pack(E_uid.T)
  Em2 = _tc_transpose_pack(E_mid.T)

  def packed_row(i):
    return jnp.bitwise_or(
        jnp.left_shift(jnp.right_shift(i, 12), 11),
        jnp.bitwise_and(i, _TBLK - 1))

  urow = packed_row(uid).reshape(B // _CHUNK, _CHUNK)
  mrow = packed_row(mid).reshape(B // _CHUNK, _CHUNK)
  upar = jnp.bitwise_and(jnp.right_shift(uid, 11), 1)
  mpar = jnp.bitwise_and(jnp.right_shift(mid, 11), 1)
  blku, blkm = _sc_gather_pair(urow, mrow, Eu2, Em2)
  return _tc_dense(gender, age, job, upar, mpar, blku, blkm,
                   E_gender, E_age, E_job, att_W, att_b)


# speedup vs baseline: 1.6753x; 1.0736x over previous
import functools

import jax
import jax.numpy as jnp
from jax import lax
from jax.experimental import pallas as pl
from jax.experimental.pallas import tpu as pltpu
from jax.experimental.pallas import tpu_sc as plsc

_NUM_WORKERS = 32
_CHUNK = 128
_TBLK = 2048


def _tc_transpose_pack(eT):
  D, N = eT.shape
  grid = pl.cdiv(N, 2 * _TBLK)
  last_blk = (N - 1) // _TBLK

  def body(xa_ref, xb_ref, o_ref):
    eye = (lax.broadcasted_iota(jnp.int32, (D, D), 0) ==
           lax.broadcasted_iota(jnp.int32, (D, D), 1)).astype(jnp.float32)
    dn = (((0,), (0,)), ((), ()))
    ya = lax.dot_general(xa_ref[...], eye, dn,
                         preferred_element_type=jnp.float32)
    yb = lax.dot_general(xb_ref[...], eye, dn,
                         preferred_element_type=jnp.float32)
    o_ref[...] = jnp.concatenate([ya, yb], axis=1)

  return pl.pallas_call(
      body,
      grid=(grid,),
      in_specs=[
          pl.BlockSpec((D, _TBLK),
                       lambda i: (0, jnp.minimum(2 * i, last_blk))),
          pl.BlockSpec((D, _TBLK),
                       lambda i: (0, jnp.minimum(2 * i + 1, last_blk))),
      ],
      out_specs=pl.BlockSpec((_TBLK, 2 * D), lambda i: (i, 0)),
      out_shape=jax.ShapeDtypeStruct((grid * _TBLK, 2 * D), jnp.float32),
  )(eT, eT)


def _sc_gather_pair(urow, mrow, Eu2, Em2):
  W = Eu2.shape[1]
  B = urow.shape[0]
  rows_w = B // _NUM_WORKERS
  nck = rows_w // _CHUNK

  mesh = plsc.VectorSubcoreMesh(core_axis_name="c", subcore_axis_name="s")

  @functools.partial(
      pl.kernel,
      out_type=(jax.ShapeDtypeStruct((B, W), jnp.float32),
                jax.ShapeDtypeStruct((B, W), jnp.float32)),
      mesh=mesh,
      compiler_params=pltpu.CompilerParams(use_tc_tiling_on_sc=False),
      scratch_types=[
          pltpu.VMEM((rows_w,), jnp.int32),
          pltpu.VMEM((rows_w,), jnp.int32),
          pltpu.VMEM((rows_w, W), jnp.float32),
          pltpu.SemaphoreType.DMA,
          pltpu.SemaphoreType.DMA,
      ],
  )
  def gather_kernel(u_hbm, m_hbm, eu_hbm, em_hbm, ou_hbm, om_hbm,
                    iu_v, im_v, rows_v, sem_u, sem_m):
    wid = lax.axis_index("s") * 2 + lax.axis_index("c")
    base = wid * rows_w
    pltpu.sync_copy(u_hbm.at[pl.ds(base, rows_w)], iu_v)
    pltpu.sync_copy(m_hbm.at[pl.ds(base, rows_w)], im_v)
    copies = []
    for j in range(nck):
      copies.append(pltpu.async_copy(
          eu_hbm.at[iu_v.at[pl.ds(j * _CHUNK, _CHUNK)]],
          rows_v.at[pl.ds(j * _CHUNK, _CHUNK)], sem_u))
    for c in copies:
      c.wait()
    pltpu.sync_copy(rows_v, ou_hbm.at[pl.ds(base, rows_w)])
    copies = []
    for j in range(nck):
      copies.append(pltpu.async_copy(
          em_hbm.at[im_v.at[pl.ds(j * _CHUNK, _CHUNK)]],
          rows_v.at[pl.ds(j * _CHUNK, _CHUNK)], sem_m))
    for c in copies:
      c.wait()
    pltpu.sync_copy(rows_v, om_hbm.at[pl.ds(base, rows_w)])

  return gather_kernel(urow, mrow, Eu2, Em2)


def _tc_dense(idx5, blku, blkm, E_g, E_a, E_j, att_W, att_b):
  B = blku.shape[0]
  D = att_W.shape[1]
  BM = 1024
  NB = B // BM

  def pad_rows(t, n):
    return jnp.concatenate(
        [t, jnp.zeros((n - t.shape[0], t.shape[1]), t.dtype)], axis=0)

  NG, NA, NJ = 8, 8, 24
  Egp = pad_rows(E_g, NG)
  Eap = pad_rows(E_a, NA)
  Ejp = pad_rows(E_j, NJ)
  b2 = att_b.reshape(1, D)

  def body(i5_ref, bu_ref, bm_ref,
           eg_ref, ea_ref, ej_ref, w_ref, b_ref, o_ref):
    idxT = jnp.transpose(i5_ref[...], (1, 0))
    g_col = idxT[:, 0:1]
    a_col = idxT[:, 1:2]
    j_col = idxT[:, 2:3]
    usel = idxT[:, 3:4] == 1
    msel = idxT[:, 4:5] == 1
    bu = bu_ref[...]
    bm = bm_ref[...]
    eu = jnp.where(usel, bu[:, D:], bu[:, :D])
    em = jnp.where(msel, bm[:, D:], bm[:, :D])
    w_top = w_ref[:D, :]
    w_bot = w_ref[D:, :]
    m = jnp.dot(em, w_top, preferred_element_type=jnp.float32) + b_ref[...]

    def attr_cell(ids, table_ref, n):
      oh = (ids == lax.broadcasted_iota(jnp.int32, (BM, n), 1)
            ).astype(jnp.float32)
      tbl = table_ref[...]
      proj = jnp.dot(tbl, w_bot, preferred_element_type=jnp.float32)
      both = jnp.dot(oh, jnp.concatenate([tbl, proj], axis=1),
                     preferred_element_type=jnp.float32)
      e_att = both[:, :D]
      v = m + both[:, D:]
      v = v - jnp.max(v, axis=1, keepdims=True)
      ex = jnp.exp(v)
      lam = ex / jnp.sum(ex, axis=1, keepdims=True)
      return lam * e_att

    cg = attr_cell(g_col, eg_ref, NG)
    ca = attr_cell(a_col, ea_ref, NA)
    cj = attr_cell(j_col, ej_ref, NJ)
    t = cg + ca + cj
    mn = cg * cg + ca * ca + cj * cj
    p_u = eu * t + 0.5 * (t * t - mn)
    o_ref[...] = jnp.sum(p_u * em, axis=1, keepdims=True)

  return pl.pallas_call(
      body,
      grid=(NB,),
      in_specs=[
          pl.BlockSpec((5, BM), lambda i: (0, i)),
          pl.BlockSpec((BM, 2 * D), lambda i: (i, 0)),
          pl.BlockSpec((BM, 2 * D), lambda i: (i, 0)),
          pl.BlockSpec((NG, D), lambda i: (0, 0)),
          pl.BlockSpec((NA, D), lambda i: (0, 0)),
          pl.BlockSpec((NJ, D), lambda i: (0, 0)),
          pl.BlockSpec((2 * D, D), lambda i: (0, 0)),
          pl.BlockSpec((1, D), lambda i: (0, 0)),
      ],
      out_specs=pl.BlockSpec((BM, 1), lambda i: (i, 0)),
      out_shape=jax.ShapeDtypeStruct((B, 1), jnp.float32),
  )(idx5, blku, blkm, Egp, Eap, Ejp, att_W, b2)


def kernel(uid, gender, age, job, mid, E_uid, E_gender, E_age, E_job, E_mid,
           att_W, att_b):
  B = uid.shape[0]
  Eu2 = _tc_transpose_pack(E_uid.T)
  Em2 = _tc_transpose_pack(E_mid.T)

  def packed_row(i):
    return jnp.bitwise_or(
        jnp.left_shift(jnp.right_shift(i, 12), 11),
        jnp.bitwise_and(i, _TBLK - 1))

  urow = packed_row(uid)
  mrow = packed_row(mid)
  upar = jnp.bitwise_and(jnp.right_shift(uid, 11), 1)
  mpar = jnp.bitwise_and(jnp.right_shift(mid, 11), 1)
  idx5 = jnp.stack([gender, age, job, upar, mpar])
  blku, blkm = _sc_gather_pair(urow, mrow, Eu2, Em2)
  return _tc_dense(idx5, blku, blkm,
                   E_gender, E_age, E_job, att_W, att_b)
